# baseline (speedup 1.0000x reference)
import functools

import jax
import jax.numpy as jnp
import numpy as np
from jax.experimental import pallas as pl
from jax.experimental.pallas import tpu as pltpu

_C1, _H1, _W1 = 64, 32, 32
_C2, _H2, _W2 = 128, 16, 16
_IMGS_PER_STEP = 8


def _resize_matrix(out_size: int, in_size: int) -> np.ndarray:
    scale = in_size / out_size
    src = (np.arange(out_size, dtype=np.float32) + 0.5) * scale - 0.5
    src = np.maximum(src, 0.0)
    i0 = np.minimum(np.floor(src).astype(np.int32), in_size - 1)
    i1 = np.minimum(i0 + 1, in_size - 1)
    frac = (src - i0.astype(np.float32)).astype(np.float32)
    rows = np.arange(out_size)
    m = np.zeros((out_size, in_size), np.float32)
    np.add.at(m, (rows, i0), 1.0 - frac)
    np.add.at(m, (rows, i1), frac)
    return m


_MT = np.kron(_resize_matrix(_H2, _H1), _resize_matrix(_W2, _W1)).T.copy()


def _body(x_ref, mt_ref, w_ref, b_ref, o_ref, *, g):
    x = x_ref[...].reshape(g * _C1, _H1 * _W1)
    pooled = jnp.dot(x, mt_ref[...],
                     preferred_element_type=jnp.float32)
    w = w_ref[...]
    b = b_ref[...]
    for i in range(g):
        y = jnp.dot(w, pooled[i * _C1:(i + 1) * _C1, :],
                    preferred_element_type=jnp.float32) + b
        o_ref[i, :, :, :] = y.reshape(_C2, _H2, _W2)


@jax.jit
def kernel(x_nchw, weight, bias):
    n = x_nchw.shape[0]
    g = _IMGS_PER_STEP if n % _IMGS_PER_STEP == 0 else 1

    mt = jnp.asarray(_MT)
    w = weight.astype(jnp.float32)
    b = bias.astype(jnp.float32).reshape(_C2, 1)

    out = pl.pallas_call(
        functools.partial(_body, g=g),
        out_shape=jax.ShapeDtypeStruct((n, _C2, _H2, _W2), x_nchw.dtype),
        grid_spec=pltpu.PrefetchScalarGridSpec(
            num_scalar_prefetch=0,
            grid=(n // g,),
            in_specs=[
                pl.BlockSpec((g, _C1, _H1, _W1), lambda i: (i, 0, 0, 0)),
                pl.BlockSpec((_H1 * _W1, _H2 * _W2), lambda i: (0, 0)),
                pl.BlockSpec((_C2, _C1), lambda i: (0, 0)),
                pl.BlockSpec((_C2, 1), lambda i: (0, 0)),
            ],
            out_specs=pl.BlockSpec((g, _C2, _H2, _W2), lambda i: (i, 0, 0, 0)),
        ),
        compiler_params=pltpu.CompilerParams(
            dimension_semantics=("parallel",),
            vmem_limit_bytes=64 << 20,
        ),
    )(x_nchw, mt, w, b)
    return out


# speedup vs baseline: 4.9001x; 4.9001x over previous
import jax
import jax.numpy as jnp
from jax.experimental import pallas as pl
from jax.experimental.pallas import tpu as pltpu

_C1, _H1, _W1 = 64, 32, 32
_C2, _H2, _W2 = 128, 16, 16
_N = 128


def _body(xe_ref, xo_ref, w_ref, b_ref, o_ref):
    a = xe_ref[:, 0] + xo_ref[:, 0]
    a2 = a.reshape(_C1, _W1 * _N)
    evens = jnp.concatenate(
        [a2[:, (2 * k) * _N:(2 * k + 1) * _N] for k in range(_W2)], axis=1)
    odds = jnp.concatenate(
        [a2[:, (2 * k + 1) * _N:(2 * k + 2) * _N] for k in range(_W2)], axis=1)
    p = (evens + odds) * 0.25
    y = jnp.dot(w_ref[...], p,
                preferred_element_type=jnp.float32) + b_ref[...]
    o_ref[:, 0] = y.reshape(_C2, _W2, _N)


@jax.jit
def kernel(x_nchw, weight, bias):
    n = x_nchw.shape[0]
    xt = jnp.transpose(x_nchw, (1, 2, 3, 0))
    w = weight.astype(jnp.float32)
    b = bias.astype(jnp.float32).reshape(_C2, 1)

    out = pl.pallas_call(
        _body,
        out_shape=jax.ShapeDtypeStruct((_C2, _H2, _W2, n), x_nchw.dtype),
        grid_spec=pltpu.PrefetchScalarGridSpec(
            num_scalar_prefetch=0,
            grid=(_H2,),
            in_specs=[
                pl.BlockSpec((_C1, 1, _W1, _N), lambda j: (0, 2 * j, 0, 0)),
                pl.BlockSpec((_C1, 1, _W1, _N), lambda j: (0, 2 * j + 1, 0, 0)),
                pl.BlockSpec((_C2, _C1), lambda j: (0, 0)),
                pl.BlockSpec((_C2, 1), lambda j: (0, 0)),
            ],
            out_specs=pl.BlockSpec((_C2, 1, _W2, _N), lambda j: (0, j, 0, 0)),
        ),
        compiler_params=pltpu.CompilerParams(
            dimension_semantics=("parallel",),
            vmem_limit_bytes=64 << 20,
        ),
    )(xt, xt, w, b)
    return jnp.transpose(out, (3, 0, 1, 2))


# speedup vs baseline: 8.8588x; 1.8079x over previous
import jax
import jax.numpy as jnp
from jax.experimental import pallas as pl
from jax.experimental.pallas import tpu as pltpu

_C1, _H1, _W1 = 64, 32, 32
_C2, _H2, _W2 = 128, 16, 16
_N = 128


def _body(xe_ref, xo_ref, wt_ref, b_ref, o_ref):
    a = xe_ref[:, 0] + xo_ref[:, 0]
    cols = []
    for k in range(_W2):
        s = a[:, 2 * k, :] + a[:, 2 * k + 1, :]
        cols.append((s * 0.25).T)
    pt = jnp.stack(cols, axis=1)
    y = jnp.dot(pt.reshape(_N * _W2, _C1), wt_ref[...],
                preferred_element_type=jnp.float32)
    o_ref[:, 0] = (y + b_ref[...]).reshape(_N, _W2, _C2)


@jax.jit
def kernel(x_nchw, weight, bias):
    n = x_nchw.shape[0]
    xt = jnp.transpose(x_nchw, (1, 2, 3, 0))
    wt = weight.astype(jnp.float32).T
    b = bias.astype(jnp.float32).reshape(1, _C2)

    out = pl.pallas_call(
        _body,
        out_shape=jax.ShapeDtypeStruct((n, _H2, _W2, _C2), x_nchw.dtype),
        grid_spec=pltpu.PrefetchScalarGridSpec(
            num_scalar_prefetch=0,
            grid=(_H2,),
            in_specs=[
                pl.BlockSpec((_C1, 1, _W1, _N), lambda j: (0, 2 * j, 0, 0)),
                pl.BlockSpec((_C1, 1, _W1, _N), lambda j: (0, 2 * j + 1, 0, 0)),
                pl.BlockSpec((_C1, _C2), lambda j: (0, 0)),
                pl.BlockSpec((1, _C2), lambda j: (0, 0)),
            ],
            out_specs=pl.BlockSpec((n, 1, _W2, _C2), lambda j: (0, j, 0, 0)),
        ),
        compiler_params=pltpu.CompilerParams(
            dimension_semantics=("parallel",),
            vmem_limit_bytes=64 << 20,
        ),
    )(xt, xt, wt, b)
    return jnp.transpose(out, (0, 3, 1, 2))


# speedup vs baseline: 9.9871x; 1.1274x over previous
import functools

import jax
import jax.numpy as jnp
from jax.experimental import pallas as pl
from jax.experimental.pallas import tpu as pltpu

_C1, _H1, _W1 = 64, 32, 32
_C2, _H2, _W2 = 128, 16, 16
_N = 128


_OH = 2


def _body(x_ref, wt_ref, b_ref, o_ref, *, oh):
    x = x_ref[...]
    cols = []
    for m in range(oh):
        a = x[:, 2 * m] + x[:, 2 * m + 1]
        for k in range(_W2):
            s = a[:, 2 * k, :] + a[:, 2 * k + 1, :]
            cols.append((s * 0.25).T)
    pt = jnp.stack(cols, axis=1)
    y = jnp.dot(pt.reshape(_N * oh * _W2, _C1), wt_ref[...],
                preferred_element_type=jnp.float32)
    o_ref[...] = (y + b_ref[...]).reshape(_N, oh, _W2, _C2)


@jax.jit
def kernel(x_nchw, weight, bias):
    n = x_nchw.shape[0]
    xt = jnp.transpose(x_nchw, (1, 2, 3, 0))
    wt = weight.astype(jnp.float32).T
    b = bias.astype(jnp.float32).reshape(1, _C2)

    out = pl.pallas_call(
        functools.partial(_body, oh=_OH),
        out_shape=jax.ShapeDtypeStruct((n, _H2, _W2, _C2), x_nchw.dtype),
        grid_spec=pltpu.PrefetchScalarGridSpec(
            num_scalar_prefetch=0,
            grid=(_H2 // _OH,),
            in_specs=[
                pl.BlockSpec((_C1, 2 * _OH, _W1, _N), lambda j: (0, j, 0, 0)),
                pl.BlockSpec((_C1, _C2), lambda j: (0, 0)),
                pl.BlockSpec((1, _C2), lambda j: (0, 0)),
            ],
            out_specs=pl.BlockSpec((n, _OH, _W2, _C2), lambda j: (0, j, 0, 0)),
        ),
        compiler_params=pltpu.CompilerParams(
            dimension_semantics=("parallel",),
            vmem_limit_bytes=64 << 20,
        ),
    )(xt, wt, b)
    return jnp.transpose(out, (0, 3, 1, 2))
